# Initial kernel scaffold; baseline (speedup 1.0000x reference)
import jax
import jax.numpy as jnp
from jax.experimental import pallas as pl


def kernel(h, edge_index, radial, coord_diff, rbf, params):
    raise NotImplementedError("write your pallas kernel here")


# speedup vs baseline: 9.6819x; 9.6819x over previous
import functools
import math

import jax
import jax.numpy as jnp
from jax import lax
from jax.experimental import pallas as pl
from jax.experimental.pallas import tpu as pltpu
from jax.experimental.pallas import tpu_sc as plsc

N = 10000
E = 160000
F = 256
HID = 256
N_HEADS = 4
RBF_K = 18
ALPHA = 2.0
EPS = 1e-8

NC = 2
NS = 16
CHUNK = 128
NCHUNKS = E // CHUNK
SCHUNK = 64
NSCHUNKS = E // SCHUNK
ATW = 128
HALF = N // NC
ACC_ROWS = 5120
ROWS_PER_SUB = ACC_ROWS // NS


def _celu(x):
    return jnp.where(x > 0, x,
                     ALPHA * (jnp.exp(jnp.minimum(x, 0.0) * (1.0 / ALPHA)) - 1.0))


def _sc_gather_body(h_hbm, row_hbm, col_hbm, src_out, tgt_out,
                    rowv, colv, srcv, tgtv, sem1, sem2):
    wid = lax.axis_index("s") * NC + lax.axis_index("c")
    nw = NC * NS

    def body(j, carry):
        ch = wid + j * nw

        @pl.when(ch < NCHUNKS)
        def _():
            base = ch * CHUNK
            pltpu.sync_copy(row_hbm.at[pl.ds(base, CHUNK)], rowv)
            pltpu.sync_copy(col_hbm.at[pl.ds(base, CHUNK)], colv)
            a = pltpu.async_copy(h_hbm.at[rowv], srcv, sem1)
            b = pltpu.async_copy(h_hbm.at[colv], tgtv, sem2)
            a.wait()
            b.wait()
            pltpu.sync_copy(srcv, src_out.at[pl.ds(base, CHUNK)])
            pltpu.sync_copy(tgtv, tgt_out.at[pl.ds(base, CHUNK)])

        return carry

    niter = (NCHUNKS + nw - 1) // nw
    lax.fori_loop(0, niter, body, 0)


def _sc_gather(h, row, col):
    mesh = plsc.VectorSubcoreMesh(core_axis_name="c", subcore_axis_name="s",
                                  num_cores=NC, num_subcores=NS)
    f = pl.kernel(
        _sc_gather_body,
        out_type=(jax.ShapeDtypeStruct((E, F), jnp.float32),
                  jax.ShapeDtypeStruct((E, F), jnp.float32)),
        mesh=mesh,
        scratch_types=(
            pltpu.VMEM((CHUNK,), jnp.int32),
            pltpu.VMEM((CHUNK,), jnp.int32),
            pltpu.VMEM((CHUNK, F), jnp.float32),
            pltpu.VMEM((CHUNK, F), jnp.float32),
            pltpu.SemaphoreType.DMA,
            pltpu.SemaphoreType.DMA,
        ),
    )
    return f(h, row, col)


EB = 1000


def _tc_edge_body(src, tgt, radial, cd, rbf,
                  wf1s, wf1t, bf1, wf2, bf2,
                  wrbf, brbf,
                  we1s, we1t, we1r, we1x, be1, we2, be2,
                  wsem1, bsem1, wsem2, bsem2,
                  wsp, bsp,
                  ef1_out, ef2_out, attn_out):
    s = src[...]
    t = tgt[...]
    r = radial[...]
    fpre = (jnp.dot(s, wf1s[...], preferred_element_type=jnp.float32)
            + jnp.dot(t, wf1t[...], preferred_element_type=jnp.float32)
            + bf1[...])
    wf = jnp.dot(_celu(fpre), wf2[...], preferred_element_type=jnp.float32) + bf2[...]
    rbf_e = jnp.dot(rbf[...], wrbf[...], preferred_element_type=jnp.float32) + brbf[...]
    x = rbf_e * wf
    epre = (jnp.dot(s, we1s[...], preferred_element_type=jnp.float32)
            + jnp.dot(t, we1t[...], preferred_element_type=jnp.float32)
            + r * we1r[...]
            + jnp.dot(x, we1x[...], preferred_element_type=jnp.float32)
            + be1[...])
    ef = _celu(jnp.dot(_celu(epre), we2[...], preferred_element_type=jnp.float32)
               + be2[...])
    semh = _celu(jnp.dot(ef, wsem1[...], preferred_element_type=jnp.float32)
                 + bsem1[...])
    sem = jnp.dot(semh, wsem2[...], preferred_element_type=jnp.float32) + bsem2[...]
    dist = jnp.sqrt(r)
    eu = 0.5 * (jnp.cos(dist * math.pi) + 1.0)
    ef = ef * (sem * eu)
    sp = jnp.dot(ef, wsp[...], preferred_element_type=jnp.float32) + bsp[...]
    c = cd[...]
    cdn = c / jnp.sqrt(jnp.sum(c * c, axis=1, keepdims=True) + 1e-12) + EPS
    ef1_out[...] = ef[:, :128]
    ef2_out[...] = ef[:, 128:]
    attn_out[...] = jnp.concatenate(
        [cdn[:, 0:1] * sp, cdn[:, 1:2] * sp, cdn[:, 2:3] * sp,
         jnp.zeros((EB, ATW - 12), jnp.float32)], axis=1)


def _tc_edge(src, tgt, radial, coord_diff, rbf, p):
    wf1 = p['Wf1']
    we1 = p['We1']
    weights = (wf1[:F], wf1[F:], p['bf1'][None, :], p['Wf2'], p['bf2'][None, :],
               p['Wrbf'], p['brbf'][None, :],
               we1[0:F], we1[F:2 * F], we1[2 * F:2 * F + 1],
               we1[2 * F + 1:], p['be1'][None, :], p['We2'], p['be2'][None, :],
               p['Wsem1'], p['bsem1'][None, :], p['Wsem2'], p['bsem2'][None, :],
               p['Wsp'], p['bsp'][None, :])
    grid = (E // EB,)

    def eb(feat):
        return pl.BlockSpec((EB, feat), lambda i: (i, 0))

    def wspec(w):
        return pl.BlockSpec(w.shape, lambda i: tuple(0 for _ in w.shape))

    return pl.pallas_call(
        _tc_edge_body,
        grid=grid,
        in_specs=[eb(F), eb(F), eb(1), eb(3), eb(RBF_K)]
                 + [wspec(w) for w in weights],
        out_specs=[eb(128), eb(128), eb(ATW)],
        out_shape=(jax.ShapeDtypeStruct((E, 128), jnp.float32),
                   jax.ShapeDtypeStruct((E, 128), jnp.float32),
                   jax.ShapeDtypeStruct((E, ATW), jnp.float32)),
    )(src, tgt, radial, coord_diff, rbf, *weights)


def _sc_scatter_body(msg_hbm, col_hbm, zero_hbm,
                     out_hbm,
                     colv, msgv, acc, sem):
    c = lax.axis_index("c")
    s = lax.axis_index("s")
    base = c * HALF
    pltpu.sync_copy(zero_hbm.at[pl.ds(0, ROWS_PER_SUB)],
                    acc.at[pl.ds(s * ROWS_PER_SUB, ROWS_PER_SUB)])
    plsc.subcore_barrier()

    def body(j, carry):
        ch = s + j * NS

        @pl.when(ch < NSCHUNKS)
        def _():
            ebase = ch * SCHUNK
            pltpu.sync_copy(col_hbm.at[pl.ds(ebase, SCHUNK)], colv)
            a = pltpu.async_copy(msg_hbm.at[pl.ds(ebase, SCHUNK)], msgv, sem)
            a.wait()
            for g in range(SCHUNK // 16):
                cv = colv[pl.ds(g * 16, 16)]
                inb = (cv >= base) & (cv < base + HALF)
                iv = jnp.where(inb, cv - base, HALF + 8 + s * 4 + (g & 3))
                pltpu.sync_copy(msgv.at[pl.ds(g * 16, 16)], acc.at[iv],
                                add=True)

        return carry

    niter = (NSCHUNKS + NS - 1) // NS
    lax.fori_loop(0, niter, body, 0)
    plsc.subcore_barrier()
    rstart = s * ROWS_PER_SUB

    @pl.when(s < NS - 1)
    def _():
        pltpu.sync_copy(acc.at[pl.ds(rstart, ROWS_PER_SUB)],
                        out_hbm.at[pl.ds(base + rstart, ROWS_PER_SUB)])

    @pl.when(s == NS - 1)
    def _():
        tail = HALF - (NS - 1) * ROWS_PER_SUB
        pltpu.sync_copy(acc.at[pl.ds(rstart, tail)],
                        out_hbm.at[pl.ds(base + rstart, tail)])


def _sc_scatter(msg, col, width):
    zero = jnp.zeros((ROWS_PER_SUB, width), jnp.float32)
    mesh = plsc.VectorSubcoreMesh(core_axis_name="c", subcore_axis_name="s",
                                  num_cores=NC, num_subcores=NS)
    f = pl.kernel(
        _sc_scatter_body,
        out_type=jax.ShapeDtypeStruct((N, width), jnp.float32),
        mesh=mesh,
        scratch_types=(
            pltpu.VMEM((SCHUNK,), jnp.int32),
            pltpu.VMEM((SCHUNK, width), jnp.float32),
            pltpu.VMEM_SHARED((ACC_ROWS, width), jnp.float32),
            pltpu.SemaphoreType.DMA,
        ),
    )
    return f(msg, col, zero)


NB = 1000


def _tc_node_body(h, agg, aggv,
                  wmu1, bmu1, wmu2, bmu2,
                  wn1h, wn1a, wn1s, bn1, wn2, bn2,
                  out):
    v = aggv[...]
    v2 = v * v
    normsq = v2[:, 0:4] + v2[:, 4:8] + v2[:, 8:12] + 1e-12
    agg_norm = jnp.sqrt(normsq)
    spat = _celu(jnp.dot(
        _celu(jnp.dot(agg_norm, wmu1[...], preferred_element_type=jnp.float32)
              + bmu1[...]),
        wmu2[...], preferred_element_type=jnp.float32) + bmu2[...])
    npre = (jnp.dot(h[...], wn1h[...], preferred_element_type=jnp.float32)
            + jnp.dot(agg[...], wn1a[...], preferred_element_type=jnp.float32)
            + jnp.dot(spat, wn1s[...], preferred_element_type=jnp.float32)
            + bn1[...])
    out[...] = _celu(jnp.dot(_celu(npre), wn2[...],
                             preferred_element_type=jnp.float32) + bn2[...])


def _tc_node(h, agg, aggv, p):
    wn1 = p['Wn1']
    weights = (p['Wmu1'], p['bmu1'][None, :], p['Wmu2'], p['bmu2'][None, :],
               wn1[0:F], wn1[F:F + HID], wn1[F + HID:], p['bn1'][None, :],
               p['Wn2'], p['bn2'][None, :])
    grid = (N // NB,)

    def nb(feat):
        return pl.BlockSpec((NB, feat), lambda i: (i, 0))

    def wspec(w):
        return pl.BlockSpec(w.shape, lambda i: tuple(0 for _ in w.shape))

    return pl.pallas_call(
        _tc_node_body,
        grid=grid,
        in_specs=[nb(F), nb(HID), nb(16)] + [wspec(w) for w in weights],
        out_specs=nb(F),
        out_shape=jax.ShapeDtypeStruct((N, F), jnp.float32),
    )(h, agg, aggv, *weights)


def kernel(h, edge_index, radial, coord_diff, rbf, params):
    row = edge_index[0]
    col = edge_index[1]
    src, tgt = _sc_gather(h, row, col)
    ef1, ef2, attnp = _tc_edge(src, tgt, radial, coord_diff, rbf, params)
    agg1 = _sc_scatter(ef1, col, 128)
    agg2 = _sc_scatter(ef2, col, 128)
    aggvp = _sc_scatter(attnp, col, ATW)
    agg = jnp.concatenate([agg1, agg2], axis=1)
    return _tc_node(h, agg, aggvp[:, :16], params)


# speedup vs baseline: 13.4903x; 1.3934x over previous
import functools
import math

import jax
import jax.numpy as jnp
from jax import lax
from jax.experimental import pallas as pl
from jax.experimental.pallas import tpu as pltpu
from jax.experimental.pallas import tpu_sc as plsc

N = 10000
E = 160000
F = 256
HID = 256
N_HEADS = 4
RBF_K = 18
ALPHA = 2.0
EPS = 1e-8

NC = 2
NS = 16
CHUNK = 128
NCHUNKS = E // CHUNK
SCHUNK = 256
NSCHUNKS = E // SCHUNK
ATW = 128
HALF = N // NC
ACC_ROWS = 5120
ROWS_PER_SUB = ACC_ROWS // NS


def _celu(x):
    return jnp.where(x > 0, x,
                     ALPHA * (jnp.exp(jnp.minimum(x, 0.0) * (1.0 / ALPHA)) - 1.0))


def _sc_gather_body(h_hbm, row_hbm, col_hbm, src_out, tgt_out,
                    rowv, colv, srcv, tgtv, sem1, sem2):
    wid = lax.axis_index("s") * NC + lax.axis_index("c")
    nw = NC * NS

    def body(j, carry):
        ch = wid + j * nw

        @pl.when(ch < NCHUNKS)
        def _():
            base = ch * CHUNK
            pltpu.sync_copy(row_hbm.at[pl.ds(base, CHUNK)], rowv)
            pltpu.sync_copy(col_hbm.at[pl.ds(base, CHUNK)], colv)
            a = pltpu.async_copy(h_hbm.at[rowv], srcv, sem1)
            b = pltpu.async_copy(h_hbm.at[colv], tgtv, sem2)
            a.wait()
            b.wait()
            pltpu.sync_copy(srcv, src_out.at[pl.ds(base, CHUNK)])
            pltpu.sync_copy(tgtv, tgt_out.at[pl.ds(base, CHUNK)])

        return carry

    niter = (NCHUNKS + nw - 1) // nw
    lax.fori_loop(0, niter, body, 0)


def _sc_gather(h, row, col):
    mesh = plsc.VectorSubcoreMesh(core_axis_name="c", subcore_axis_name="s",
                                  num_cores=NC, num_subcores=NS)
    f = pl.kernel(
        _sc_gather_body,
        out_type=(jax.ShapeDtypeStruct((E, F), jnp.float32),
                  jax.ShapeDtypeStruct((E, F), jnp.float32)),
        mesh=mesh,
        scratch_types=(
            pltpu.VMEM((CHUNK,), jnp.int32),
            pltpu.VMEM((CHUNK,), jnp.int32),
            pltpu.VMEM((CHUNK, F), jnp.float32),
            pltpu.VMEM((CHUNK, F), jnp.float32),
            pltpu.SemaphoreType.DMA,
            pltpu.SemaphoreType.DMA,
        ),
    )
    return f(h, row, col)


EB = 1000


def _tc_edge_body(src, tgt, radial, cd, rbf,
                  wf1s, wf1t, bf1, wf2, bf2,
                  wrbf, brbf,
                  we1s, we1t, we1r, we1x, be1, we2, be2,
                  wsem1, bsem1, wsem2, bsem2,
                  wsp, bsp,
                  ef1_out, ef2_out, attn_out):
    s = src[...].astype(jnp.bfloat16)
    t = tgt[...].astype(jnp.bfloat16)
    r = radial[...]
    fpre = (jnp.dot(s, wf1s[...], preferred_element_type=jnp.float32)
            + jnp.dot(t, wf1t[...], preferred_element_type=jnp.float32)
            + bf1[...])
    wf = jnp.dot(_celu(fpre).astype(jnp.bfloat16), wf2[...],
                 preferred_element_type=jnp.float32) + bf2[...]
    rbf_e = jnp.dot(rbf[...], wrbf[...], preferred_element_type=jnp.float32) + brbf[...]
    x = (rbf_e * wf).astype(jnp.bfloat16)
    epre = (jnp.dot(s, we1s[...], preferred_element_type=jnp.float32)
            + jnp.dot(t, we1t[...], preferred_element_type=jnp.float32)
            + r * we1r[...]
            + jnp.dot(x, we1x[...], preferred_element_type=jnp.float32)
            + be1[...])
    ef = _celu(jnp.dot(_celu(epre).astype(jnp.bfloat16), we2[...],
                       preferred_element_type=jnp.float32)
               + be2[...])
    semh = _celu(jnp.dot(ef, wsem1[...], preferred_element_type=jnp.float32)
                 + bsem1[...])
    sem = jnp.dot(semh, wsem2[...], preferred_element_type=jnp.float32) + bsem2[...]
    dist = jnp.sqrt(r)
    eu = 0.5 * (jnp.cos(dist * math.pi) + 1.0)
    ef = ef * (sem * eu)
    sp = jnp.dot(ef, wsp[...], preferred_element_type=jnp.float32) + bsp[...]
    c = cd[...]
    cdn = c / jnp.sqrt(jnp.sum(c * c, axis=1, keepdims=True) + 1e-12) + EPS
    ef1_out[...] = ef[:, :128]
    ef2_out[...] = ef[:, 128:]
    attn_out[...] = jnp.concatenate(
        [cdn[:, 0:1] * sp, cdn[:, 1:2] * sp, cdn[:, 2:3] * sp,
         jnp.zeros((EB, ATW - 12), jnp.float32)], axis=1)


def _tc_edge(src, tgt, radial, coord_diff, rbf, p):
    wf1 = p['Wf1']
    we1 = p['We1']
    bf16 = jnp.bfloat16
    weights = (wf1[:F].astype(bf16), wf1[F:].astype(bf16), p['bf1'][None, :],
               p['Wf2'].astype(bf16), p['bf2'][None, :],
               p['Wrbf'], p['brbf'][None, :],
               we1[0:F].astype(bf16), we1[F:2 * F].astype(bf16),
               we1[2 * F:2 * F + 1],
               we1[2 * F + 1:].astype(bf16), p['be1'][None, :],
               p['We2'].astype(bf16), p['be2'][None, :],
               p['Wsem1'], p['bsem1'][None, :], p['Wsem2'], p['bsem2'][None, :],
               p['Wsp'], p['bsp'][None, :])
    grid = (E // EB,)

    def eb(feat):
        return pl.BlockSpec((EB, feat), lambda i: (i, 0))

    def wspec(w):
        return pl.BlockSpec(w.shape, lambda i: tuple(0 for _ in w.shape))

    return pl.pallas_call(
        _tc_edge_body,
        grid=grid,
        in_specs=[eb(F), eb(F), eb(1), eb(3), eb(RBF_K)]
                 + [wspec(w) for w in weights],
        out_specs=[eb(128), eb(128), eb(ATW)],
        out_shape=(jax.ShapeDtypeStruct((E, 128), jnp.float32),
                   jax.ShapeDtypeStruct((E, 128), jnp.float32),
                   jax.ShapeDtypeStruct((E, ATW), jnp.float32)),
    )(src, tgt, radial, coord_diff, rbf, *weights)


def _sc_scatter_body(msg_hbm, col_hbm, zero_hbm,
                     out_hbm,
                     colv0, colv1, msgv0, msgv1, acc,
                     lsem0, lsem1, ssem):
    c = lax.axis_index("c")
    s = lax.axis_index("s")
    base = c * HALF
    pltpu.sync_copy(zero_hbm.at[pl.ds(0, ROWS_PER_SUB)],
                    acc.at[pl.ds(s * ROWS_PER_SUB, ROWS_PER_SUB)])
    plsc.subcore_barrier()

    colvs = (colv0, colv1)
    msgvs = (msgv0, msgv1)
    lsems = (lsem0, lsem1)
    niter = (NSCHUNKS + NS - 1) // NS

    def load(j, b):
        ch = s + j * NS

        @pl.when(ch < NSCHUNKS)
        def _():
            ebase = ch * SCHUNK
            pltpu.async_copy(col_hbm.at[pl.ds(ebase, SCHUNK)], colvs[b],
                             lsems[b])
            pltpu.async_copy(msg_hbm.at[pl.ds(ebase, SCHUNK)], msgvs[b],
                             lsems[b])

    load(0, 0)

    def body(j, carry):
        ch = s + j * NS
        for b in range(2):

            @pl.when((j & 1) == b)
            def _():
                @pl.when(ch < NSCHUNKS)
                def _():
                    pltpu.make_async_copy(
                        col_hbm.at[pl.ds(0, SCHUNK)], colvs[b], lsems[b]).wait()
                    pltpu.make_async_copy(
                        msg_hbm.at[pl.ds(0, SCHUNK)], msgvs[b], lsems[b]).wait()
                load(j + 1, 1 - b)

                @pl.when(ch < NSCHUNKS)
                def _():
                    descs = []
                    for g in range(SCHUNK // 16):
                        cv = colvs[b][pl.ds(g * 16, 16)]
                        inb = (cv >= base) & (cv < base + HALF)
                        iv = jnp.where(inb, cv - base,
                                       HALF + 8 + s * 4 + (g & 7))
                        descs.append(pltpu.async_copy(
                            msgvs[b].at[pl.ds(g * 16, 16)], acc.at[iv],
                            ssem, add=True))
                    for d in descs:
                        d.wait()

        return carry

    lax.fori_loop(0, niter, body, 0, unroll=2)
    plsc.subcore_barrier()
    rstart = s * ROWS_PER_SUB

    @pl.when(s < NS - 1)
    def _():
        pltpu.sync_copy(acc.at[pl.ds(rstart, ROWS_PER_SUB)],
                        out_hbm.at[pl.ds(base + rstart, ROWS_PER_SUB)])

    @pl.when(s == NS - 1)
    def _():
        tail = HALF - (NS - 1) * ROWS_PER_SUB
        pltpu.sync_copy(acc.at[pl.ds(rstart, tail)],
                        out_hbm.at[pl.ds(base + rstart, tail)])


def _sc_scatter(msg, col, width):
    zero = jnp.zeros((ROWS_PER_SUB, width), jnp.float32)
    mesh = plsc.VectorSubcoreMesh(core_axis_name="c", subcore_axis_name="s",
                                  num_cores=NC, num_subcores=NS)
    f = pl.kernel(
        _sc_scatter_body,
        out_type=jax.ShapeDtypeStruct((N, width), jnp.float32),
        mesh=mesh,
        scratch_types=(
            pltpu.VMEM((SCHUNK,), jnp.int32),
            pltpu.VMEM((SCHUNK,), jnp.int32),
            pltpu.VMEM((SCHUNK, width), jnp.float32),
            pltpu.VMEM((SCHUNK, width), jnp.float32),
            pltpu.VMEM_SHARED((ACC_ROWS, width), jnp.float32),
            pltpu.SemaphoreType.DMA,
            pltpu.SemaphoreType.DMA,
            pltpu.SemaphoreType.DMA,
        ),
    )
    return f(msg, col, zero)


NB = 1000


def _tc_node_body(h, agg, aggv,
                  wmu1, bmu1, wmu2, bmu2,
                  wn1h, wn1a, wn1s, bn1, wn2, bn2,
                  out):
    v = aggv[...]
    v2 = v * v
    normsq = v2[:, 0:4] + v2[:, 4:8] + v2[:, 8:12] + 1e-12
    agg_norm = jnp.sqrt(normsq)
    spat = _celu(jnp.dot(
        _celu(jnp.dot(agg_norm, wmu1[...], preferred_element_type=jnp.float32)
              + bmu1[...]),
        wmu2[...], preferred_element_type=jnp.float32) + bmu2[...])
    npre = (jnp.dot(h[...], wn1h[...], preferred_element_type=jnp.float32)
            + jnp.dot(agg[...], wn1a[...], preferred_element_type=jnp.float32)
            + jnp.dot(spat, wn1s[...], preferred_element_type=jnp.float32)
            + bn1[...])
    out[...] = _celu(jnp.dot(_celu(npre), wn2[...],
                             preferred_element_type=jnp.float32) + bn2[...])


def _tc_node(h, agg, aggv, p):
    wn1 = p['Wn1']
    weights = (p['Wmu1'], p['bmu1'][None, :], p['Wmu2'], p['bmu2'][None, :],
               wn1[0:F], wn1[F:F + HID], wn1[F + HID:], p['bn1'][None, :],
               p['Wn2'], p['bn2'][None, :])
    grid = (N // NB,)

    def nb(feat):
        return pl.BlockSpec((NB, feat), lambda i: (i, 0))

    def wspec(w):
        return pl.BlockSpec(w.shape, lambda i: tuple(0 for _ in w.shape))

    return pl.pallas_call(
        _tc_node_body,
        grid=grid,
        in_specs=[nb(F), nb(HID), nb(16)] + [wspec(w) for w in weights],
        out_specs=nb(F),
        out_shape=jax.ShapeDtypeStruct((N, F), jnp.float32),
    )(h, agg, aggv, *weights)


def kernel(h, edge_index, radial, coord_diff, rbf, params):
    row = edge_index[0]
    col = edge_index[1]
    src, tgt = _sc_gather(h, row, col)
    ef1, ef2, attnp = _tc_edge(src, tgt, radial, coord_diff, rbf, params)
    agg1 = _sc_scatter(ef1, col, 128)
    agg2 = _sc_scatter(ef2, col, 128)
    aggvp = _sc_scatter(attnp, col, ATW)
    agg = jnp.concatenate([agg1, agg2], axis=1)
    return _tc_node(h, agg, aggvp[:, :16], params)
